# Initial kernel scaffold; baseline (speedup 1.0000x reference)
import jax
import jax.numpy as jnp
from jax.experimental import pallas as pl


def kernel(points, sa1_w, sa1_b, sa2_w, sa2_b, fp1_w, fp1_b, fp2_w, fp2_b, fc_w, fc_b):
    raise NotImplementedError("write your pallas kernel here")


# speedup vs baseline: 12.2483x; 12.2483x over previous
import functools

import numpy as np
import jax
import jax.numpy as jnp
from jax import lax
from jax.experimental import pallas as pl
from jax.experimental.pallas import tpu as pltpu
from jax.experimental.pallas import tpu_sc as plsc

_F32 = jnp.float32
_I32 = jnp.int32


def _fps_body(x_ref, y_ref, z_ref, idx_ref, cx_ref, cy_ref, cz_ref, *, npoint):
    x = x_ref[...]
    y = y_ref[...]
    z = z_ref[...]
    B, N = x.shape
    lane = lax.broadcasted_iota(_I32, (B, N), 1).astype(_F32)
    piota = lax.broadcasted_iota(_I32, (B, npoint), 1).astype(_F32)

    def body(i, state):
        idx, cxa, cya, cza, dist, far = state
        sel = lane == far
        cx = jnp.sum(jnp.where(sel, x, 0.0), axis=1, keepdims=True)
        cy = jnp.sum(jnp.where(sel, y, 0.0), axis=1, keepdims=True)
        cz = jnp.sum(jnp.where(sel, z, 0.0), axis=1, keepdims=True)
        slot = jnp.where(piota == i.astype(_F32), 1.0, 0.0)
        idx = idx + slot * far
        cxa = cxa + slot * cx
        cya = cya + slot * cy
        cza = cza + slot * cz
        dx = x - cx
        dy = y - cy
        dz = z - cz
        d = (dx * dx + dy * dy) + dz * dz
        dist = jnp.minimum(dist, d)
        m = jnp.max(dist, axis=1, keepdims=True)
        far = jnp.min(jnp.where(dist == m, lane, np.float32(N)),
                      axis=1, keepdims=True)
        return idx, cxa, cya, cza, dist, far

    zp = (piota * 0.0
          + lax.broadcasted_iota(_I32, (B, npoint), 0).astype(_F32) * 0.0)
    zn = (lane * 0.0
          + lax.broadcasted_iota(_I32, (B, N), 0).astype(_F32) * 0.0)
    init = (zp, zp, zp, zp, zn + 1e10, jnp.zeros((B, 1), _F32))
    idx, cxa, cya, cza, _, _ = lax.fori_loop(0, npoint, body, init)
    idx_ref[...] = idx.astype(_I32)
    cx_ref[...] = cxa
    cy_ref[...] = cya
    cz_ref[...] = cza


def _fps(x, y, z, npoint):
    B, N = x.shape
    full = lambda s: pl.BlockSpec(s, lambda: (0,) * len(s))
    outs = pl.pallas_call(
        functools.partial(_fps_body, npoint=npoint),
        in_specs=[full((B, N))] * 3,
        out_specs=[full((B, npoint))] * 4,
        out_shape=[jax.ShapeDtypeStruct((B, npoint), _I32)] +
                  [jax.ShapeDtypeStruct((B, npoint), _F32)] * 3,
    )(x, y, z)
    return outs


def _cumsum_lanes(m):
    c = m
    sh = 1
    n = m.shape[-1]
    while sh < n:
        z = jnp.zeros(m.shape[:-1] + (sh,), m.dtype)
        c = c + jnp.concatenate([z, c[..., :-sh]], axis=-1)
        sh *= 2
    return c


def _ballq_body(q_ref, pt_ref, out_ref, *, thr, K, N):
    b = pl.program_id(0)
    q = q_ref[0]
    pt = pt_ref[0]
    qx = q[:, 0:1]
    qy = q[:, 1:2]
    qz = q[:, 2:3]
    qn = (qx * qx + qy * qy) + qz * qz
    pn = jnp.sum(pt * pt, axis=0, keepdims=True)
    cross = jnp.dot(q, pt)
    sqd = (qn + pn) - 2.0 * cross
    mask = jnp.where(sqd > thr, 0.0, 1.0)
    c = _cumsum_lanes(mask)
    cnt = c[:, N - 1:N]
    cols = []
    for k in range(K):
        cols.append(jnp.sum(jnp.where(c <= np.float32(k), 1.0, 0.0),
                            axis=1, keepdims=True))
    pk = jnp.concatenate(cols, axis=1)
    ST = q.shape[0]
    kio = lax.broadcasted_iota(_I32, (ST, K), 1)
    grp = jnp.where(kio < cnt.astype(_I32), pk, cols[0])
    out_ref[...] = (grp.astype(_I32) + b * N)[None]


def _ballq(q, pt, thr, K, stile):
    B, S, _ = q.shape
    N = pt.shape[-1]
    grid = (B, S // stile)
    out = pl.pallas_call(
        functools.partial(_ballq_body, thr=thr, K=K, N=N),
        grid=grid,
        in_specs=[pl.BlockSpec((1, stile, 3), lambda b, t: (b, t, 0)),
                  pl.BlockSpec((1, 3, N), lambda b, t: (b, 0, 0))],
        out_specs=pl.BlockSpec((1, stile, K), lambda b, t: (b, t, 0)),
        out_shape=jax.ShapeDtypeStruct((B, S, K), _I32),
    )(q, pt)
    return out


@functools.lru_cache(maxsize=None)
def _make_sc_gather(V, D, Btot):
    NC, NS = 2, 16
    NW = NC * NS
    CH = 128
    bw = Btot // NW
    nch = bw // CH
    mesh = plsc.VectorSubcoreMesh(core_axis_name="c", subcore_axis_name="s")

    @functools.partial(
        pl.kernel,
        mesh=mesh,
        out_type=jax.ShapeDtypeStruct((Btot, D), _F32),
        scratch_types=[pltpu.VMEM((CH,), _I32),
                       pltpu.VMEM((CH, D), _F32),
                       pltpu.SemaphoreType.DMA],
        compiler_params=pltpu.CompilerParams(use_tc_tiling_on_sc=False),
    )
    def k(table_hbm, idx_hbm, out_hbm, idx_v, rows_v, sem):
        wid = lax.axis_index("s") * NC + lax.axis_index("c")
        base = wid * bw

        def chunk(j, carry):
            off = base + j * CH
            pltpu.sync_copy(idx_hbm.at[pl.ds(off, CH)], idx_v)
            pltpu.async_copy(table_hbm.at[idx_v], rows_v, sem).wait()
            pltpu.sync_copy(rows_v, out_hbm.at[pl.ds(off, CH)])
            return carry

        lax.fori_loop(0, nch, chunk, 0)

    return k


def _sc_gather_rows(table, idx):
    V, D = table.shape
    return _make_sc_gather(V, D, idx.shape[0])(table, idx)


def _mlp_pool_body(g_ref, off_ref, w1_ref, b1_ref, w2_ref, b2_ref,
                   w3_ref, b3_ref, out_ref, *, stile, K, cout):
    g = g_ref[...] - off_ref[...]
    h = jnp.maximum(jnp.dot(g, w1_ref[...], preferred_element_type=_F32)
                    + b1_ref[...], 0.0)
    h = jnp.maximum(jnp.dot(h, w2_ref[...], preferred_element_type=_F32)
                    + b2_ref[...], 0.0)
    h = jnp.maximum(jnp.dot(h, w3_ref[...], preferred_element_type=_F32)
                    + b3_ref[...], 0.0)
    h = h.reshape(stile, K, cout)
    out_ref[...] = jnp.max(h, axis=1)[None]


def _mlp_pool(gath, off, ws, bs, B, S, K, stile):
    rows = stile * K
    din = gath.shape[1]
    cout = ws[2].shape[1]
    grid = (B, S // stile)
    nblk = S // stile
    full = lambda a: pl.BlockSpec(a.shape, lambda b, t: (0,) * a.ndim)
    b2d = [b.reshape(1, -1) for b in bs]
    out = pl.pallas_call(
        functools.partial(_mlp_pool_body, stile=stile, K=K, cout=cout),
        grid=grid,
        in_specs=[pl.BlockSpec((rows, din), lambda b, t, n=nblk: (b * n + t, 0)),
                  pl.BlockSpec((rows, din), lambda b, t, n=nblk: (b * n + t, 0)),
                  full(ws[0]), full(b2d[0]), full(ws[1]), full(b2d[1]),
                  full(ws[2]), full(b2d[2])],
        out_specs=pl.BlockSpec((1, stile, cout), lambda b, t: (b, t, 0)),
        out_shape=jax.ShapeDtypeStruct((B, S, cout), _F32),
    )(gath, off, ws[0], b2d[0], ws[1], b2d[1], ws[2], b2d[2])
    return out


def _nn3_weights(q, pt):
    M = q.shape[0]
    S = pt.shape[1]
    qx, qy, qz = q[:, 0:1], q[:, 1:2], q[:, 2:3]
    qn = (qx * qx + qy * qy) + qz * qz
    pn = jnp.sum(pt * pt, axis=0, keepdims=True)
    cross = jnp.dot(q, pt)
    d = (qn + pn) - 2.0 * cross
    lio = lax.broadcasted_iota(_I32, (M, S), 1)
    ohs, wts = [], []
    for _ in range(3):
        m = jnp.min(d, axis=1, keepdims=True)
        a = jnp.min(jnp.where(d == m, lio, S), axis=1, keepdims=True)
        oh = lio == a
        wts.append(1.0 / jnp.maximum(m, 1e-10))
        ohs.append(oh)
        d = jnp.where(oh, jnp.float32(np.inf), d)
    wsum = (wts[0] + wts[1]) + wts[2]
    wm = (jnp.where(ohs[0], wts[0] / wsum, 0.0)
          + jnp.where(ohs[1], wts[1] / wsum, 0.0)
          + jnp.where(ohs[2], wts[2] / wsum, 0.0))
    return wm


def _fp1_body(q_ref, pt_ref, f1_ref, f2_ref, w1a_ref, w1b_ref, b1_ref,
              w2_ref, b2_ref, out_ref):
    wm = _nn3_weights(q_ref[0], pt_ref[0])
    interp = jnp.dot(wm, f2_ref[0], preferred_element_type=_F32)
    h = jnp.maximum(
        jnp.dot(f1_ref[0], w1a_ref[...], preferred_element_type=_F32)
        + jnp.dot(interp, w1b_ref[...], preferred_element_type=_F32)
        + b1_ref[...], 0.0)
    h = jnp.maximum(jnp.dot(h, w2_ref[...], preferred_element_type=_F32)
                    + b2_ref[...], 0.0)
    out_ref[...] = h[None]


def _fp1(xyz1, xyz2t, nf1, nf2, ws, bs):
    B, S1, _ = xyz1.shape
    S2 = xyz2t.shape[2]
    c1 = nf1.shape[2]
    c2 = nf2.shape[2]
    w1a, w1b = ws[0][:c1], ws[0][c1:]
    b2d = [b.reshape(1, -1) for b in bs]
    full = lambda a: pl.BlockSpec(a.shape, lambda b: (0,) * a.ndim)
    out = pl.pallas_call(
        _fp1_body,
        grid=(B,),
        in_specs=[pl.BlockSpec((1, S1, 3), lambda b: (b, 0, 0)),
                  pl.BlockSpec((1, 3, S2), lambda b: (b, 0, 0)),
                  pl.BlockSpec((1, S1, c1), lambda b: (b, 0, 0)),
                  pl.BlockSpec((1, S2, c2), lambda b: (b, 0, 0)),
                  full(w1a), full(w1b), full(b2d[0]),
                  full(ws[1]), full(b2d[1])],
        out_specs=pl.BlockSpec((1, S1, ws[1].shape[1]), lambda b: (b, 0, 0)),
        out_shape=jax.ShapeDtypeStruct((B, S1, ws[1].shape[1]), _F32),
    )(xyz1, xyz2t, nf1, nf2, w1a, w1b, b2d[0], ws[1], b2d[1])
    return out


def _fp2fc_body(q_ref, pt_ref, up_ref, w1a_ref, w1b_ref, b1_ref, w2_ref,
                b2_ref, w3_ref, b3_ref, wc1_ref, bc1_ref, wc2_ref, bc2_ref,
                out_ref):
    q = q_ref[0]
    wm = _nn3_weights(q, pt_ref[0])
    interp = jnp.dot(wm, up_ref[0], preferred_element_type=_F32)
    h = jnp.maximum(
        jnp.dot(q, w1a_ref[...], preferred_element_type=_F32)
        + jnp.dot(interp, w1b_ref[...], preferred_element_type=_F32)
        + b1_ref[...], 0.0)
    h = jnp.maximum(jnp.dot(h, w2_ref[...], preferred_element_type=_F32)
                    + b2_ref[...], 0.0)
    h = jnp.maximum(jnp.dot(h, w3_ref[...], preferred_element_type=_F32)
                    + b3_ref[...], 0.0)
    h = jnp.maximum(jnp.dot(h, wc1_ref[...], preferred_element_type=_F32)
                    + bc1_ref[...], 0.0)
    out_ref[...] = (jnp.dot(h, wc2_ref[...], preferred_element_type=_F32)
                    + bc2_ref[...])[None]


def _fp2fc(xyz, xyz1t, up1, fp2_w, fp2_b, fc_w, fc_b, ntile):
    B, N, _ = xyz.shape
    S = xyz1t.shape[2]
    cu = up1.shape[2]
    ncls = fc_w[1].shape[1]
    w1a, w1b = fp2_w[0][:3], fp2_w[0][3:]
    b2d = [b.reshape(1, -1) for b in fp2_b] + [b.reshape(1, -1) for b in fc_b]
    full = lambda a: pl.BlockSpec(a.shape, lambda b, t: (0,) * a.ndim)
    out = pl.pallas_call(
        _fp2fc_body,
        grid=(B, N // ntile),
        in_specs=[pl.BlockSpec((1, ntile, 3), lambda b, t: (b, t, 0)),
                  pl.BlockSpec((1, 3, S), lambda b, t: (b, 0, 0)),
                  pl.BlockSpec((1, S, cu), lambda b, t: (b, 0, 0)),
                  full(w1a), full(w1b), full(b2d[0]),
                  full(fp2_w[1]), full(b2d[1]),
                  full(fp2_w[2]), full(b2d[2]),
                  full(fc_w[0]), full(b2d[3]),
                  full(fc_w[1]), full(b2d[4])],
        out_specs=pl.BlockSpec((1, ntile, ncls), lambda b, t: (b, t, 0)),
        out_shape=jax.ShapeDtypeStruct((B, N, ncls), _F32),
    )(xyz, xyz1t, up1, w1a, w1b, b2d[0], fp2_w[1], b2d[1], fp2_w[2], b2d[2],
      fc_w[0], b2d[3], fc_w[1], b2d[4])
    return out


def kernel(points, sa1_w, sa1_b, sa2_w, sa2_b, fp1_w, fp1_b, fp2_w, fp2_b,
           fc_w, fc_b):
    xyz = points[:, :, :3]
    B, N, _ = xyz.shape
    xt = jnp.transpose(xyz, (0, 2, 1))
    x2, y2, z2 = xt[:, 0], xt[:, 1], xt[:, 2]

    S1, K1 = 512, 32
    _, cx1, cy1, cz1 = _fps(x2, y2, z2, S1)
    new_xyz1 = jnp.stack([cx1, cy1, cz1], axis=-1)
    gidx1 = _ballq(new_xyz1, xt, np.float32(0.2 * 0.2), K1, 128)
    table1 = jnp.concatenate(
        [xyz, xyz, jnp.zeros((B, N, 10), _F32)], axis=-1).reshape(B * N, 16)
    gath1 = _sc_gather_rows(table1, gidx1.reshape(-1))
    off1 = jnp.repeat(
        jnp.concatenate([new_xyz1, jnp.zeros((B, S1, 13), _F32)],
                        axis=-1).reshape(B * S1, 16), K1, axis=0)
    w1p = jnp.concatenate([sa1_w[0], jnp.zeros((10, sa1_w[0].shape[1]), _F32)],
                          axis=0)
    nf1 = _mlp_pool(gath1, off1, [w1p, sa1_w[1], sa1_w[2]], sa1_b,
                    B, S1, K1, 64)

    S2, K2 = 128, 64
    _, cx2, cy2, cz2 = _fps(cx1, cy1, cz1, S2)
    new_xyz2 = jnp.stack([cx2, cy2, cz2], axis=-1)
    xyz1t = jnp.stack([cx1, cy1, cz1], axis=1)
    gidx2 = _ballq(new_xyz2, xyz1t, np.float32(0.4 * 0.4), K2, S2)
    c1 = nf1.shape[2]
    din2 = 3 + c1
    pad2 = (-din2) % 16
    table2 = jnp.concatenate(
        [new_xyz1, nf1, jnp.zeros((B, S1, pad2), _F32)],
        axis=-1).reshape(B * S1, din2 + pad2)
    gath2 = _sc_gather_rows(table2, gidx2.reshape(-1))
    off2 = jnp.repeat(
        jnp.concatenate([new_xyz2, jnp.zeros((B, S2, din2 + pad2 - 3), _F32)],
                        axis=-1).reshape(B * S2, din2 + pad2), K2, axis=0)
    w2p = jnp.concatenate([sa2_w[0], jnp.zeros((pad2, sa2_w[0].shape[1]),
                                               _F32)], axis=0)
    nf2 = _mlp_pool(gath2, off2, [w2p, sa2_w[1], sa2_w[2]], sa2_b,
                    B, S2, K2, 32)

    xyz2t = jnp.stack([cx2, cy2, cz2], axis=1)
    up1 = _fp1(new_xyz1, xyz2t, nf1, nf2, fp1_w, fp1_b)

    logits = _fp2fc(xyz, xyz1t, up1, fp2_w, fp2_b, fc_w, fc_b, 1024)
    return logits


# speedup vs baseline: 13.0108x; 1.0623x over previous
import functools

import numpy as np
import jax
import jax.numpy as jnp
from jax import lax
from jax.experimental import pallas as pl
from jax.experimental.pallas import tpu as pltpu
from jax.experimental.pallas import tpu_sc as plsc

_F32 = jnp.float32
_I32 = jnp.int32


def _fps_body(x_ref, y_ref, z_ref, idx_ref, cx_ref, cy_ref, cz_ref, *, npoint):
    x = x_ref[...]
    y = y_ref[...]
    z = z_ref[...]
    B, N = x.shape
    lane = lax.broadcasted_iota(_I32, (B, N), 1).astype(_F32)
    piota = lax.broadcasted_iota(_I32, (B, npoint), 1).astype(_F32)

    def body(i, state):
        idx, cxa, cya, cza, dist, far = state
        sel = lane == far
        cx = jnp.sum(jnp.where(sel, x, 0.0), axis=1, keepdims=True)
        cy = jnp.sum(jnp.where(sel, y, 0.0), axis=1, keepdims=True)
        cz = jnp.sum(jnp.where(sel, z, 0.0), axis=1, keepdims=True)
        slot = jnp.where(piota == i.astype(_F32), 1.0, 0.0)
        idx = idx + slot * far
        cxa = cxa + slot * cx
        cya = cya + slot * cy
        cza = cza + slot * cz
        dx = x - cx
        dy = y - cy
        dz = z - cz
        d = (dx * dx + dy * dy) + dz * dz
        dist = jnp.minimum(dist, d)
        m = jnp.max(dist, axis=1, keepdims=True)
        far = jnp.min(jnp.where(dist == m, lane, np.float32(N)),
                      axis=1, keepdims=True)
        return idx, cxa, cya, cza, dist, far

    zp = (piota * 0.0
          + lax.broadcasted_iota(_I32, (B, npoint), 0).astype(_F32) * 0.0)
    zn = (lane * 0.0
          + lax.broadcasted_iota(_I32, (B, N), 0).astype(_F32) * 0.0)
    init = (zp, zp, zp, zp, zn + 1e10, jnp.zeros((B, 1), _F32))
    idx, cxa, cya, cza, _, _ = lax.fori_loop(0, npoint, body, init)
    idx_ref[...] = idx.astype(_I32)
    cx_ref[...] = cxa
    cy_ref[...] = cya
    cz_ref[...] = cza


def _fps(x, y, z, npoint):
    B, N = x.shape
    full = lambda s: pl.BlockSpec(s, lambda: (0,) * len(s))
    outs = pl.pallas_call(
        functools.partial(_fps_body, npoint=npoint),
        in_specs=[full((B, N))] * 3,
        out_specs=[full((B, npoint))] * 4,
        out_shape=[jax.ShapeDtypeStruct((B, npoint), _I32)] +
                  [jax.ShapeDtypeStruct((B, npoint), _F32)] * 3,
    )(x, y, z)
    return outs


def _cumsum_lanes(m):
    c = m
    sh = 1
    n = m.shape[-1]
    while sh < n:
        z = jnp.zeros(m.shape[:-1] + (sh,), m.dtype)
        c = c + jnp.concatenate([z, c[..., :-sh]], axis=-1)
        sh *= 2
    return c


def _ballq_body(q_ref, pt_ref, out_ref, *, thr, K, N):
    b = pl.program_id(0)
    q = q_ref[0]
    pt = pt_ref[0]
    qx = q[:, 0:1]
    qy = q[:, 1:2]
    qz = q[:, 2:3]
    qn = (qx * qx + qy * qy) + qz * qz
    pn = jnp.sum(pt * pt, axis=0, keepdims=True)
    cross = jnp.dot(q, pt)
    sqd = (qn + pn) - 2.0 * cross
    mask = jnp.where(sqd > thr, 0.0, 1.0)
    c = _cumsum_lanes(mask)
    cnt = c[:, N - 1:N]
    cols = []
    for k in range(K):
        cols.append(jnp.sum(jnp.where(c <= np.float32(k), 1.0, 0.0),
                            axis=1, keepdims=True))
    pk = jnp.concatenate(cols, axis=1)
    ST = q.shape[0]
    kio = lax.broadcasted_iota(_I32, (ST, K), 1)
    grp = jnp.where(kio < cnt.astype(_I32), pk, cols[0])
    out_ref[...] = (grp.astype(_I32) + b * N)[None]


def _ballq(q, pt, thr, K, stile):
    B, S, _ = q.shape
    N = pt.shape[-1]
    grid = (B, S // stile)
    out = pl.pallas_call(
        functools.partial(_ballq_body, thr=thr, K=K, N=N),
        grid=grid,
        in_specs=[pl.BlockSpec((1, stile, 3), lambda b, t: (b, t, 0)),
                  pl.BlockSpec((1, 3, N), lambda b, t: (b, 0, 0))],
        out_specs=pl.BlockSpec((1, stile, K), lambda b, t: (b, t, 0)),
        out_shape=jax.ShapeDtypeStruct((B, S, K), _I32),
    )(q, pt)
    return out


@functools.lru_cache(maxsize=None)
def _make_sc_gather(V, D, Btot):
    NC, NS = 2, 16
    NW = NC * NS
    CH = 128
    bw = Btot // NW
    nch = bw // CH
    mesh = plsc.VectorSubcoreMesh(core_axis_name="c", subcore_axis_name="s")

    @functools.partial(
        pl.kernel,
        mesh=mesh,
        out_type=jax.ShapeDtypeStruct((Btot, D), _F32),
        scratch_types=[pltpu.VMEM((CH,), _I32),
                       pltpu.VMEM((CH, D), _F32),
                       pltpu.SemaphoreType.DMA],
        compiler_params=pltpu.CompilerParams(use_tc_tiling_on_sc=False),
    )
    def k(table_hbm, idx_hbm, out_hbm, idx_v, rows_v, sem):
        wid = lax.axis_index("s") * NC + lax.axis_index("c")
        base = wid * bw

        def chunk(j, carry):
            off = base + j * CH
            pltpu.sync_copy(idx_hbm.at[pl.ds(off, CH)], idx_v)
            pltpu.async_copy(table_hbm.at[idx_v], rows_v, sem).wait()
            pltpu.sync_copy(rows_v, out_hbm.at[pl.ds(off, CH)])
            return carry

        lax.fori_loop(0, nch, chunk, 0)

    return k


def _sc_gather_rows(table, idx):
    V, D = table.shape
    return _make_sc_gather(V, D, idx.shape[0])(table, idx)


def _mlp_pool_body(g_ref, nx_ref, w1o_ref, w1_ref, b1_ref, w2_ref, b2_ref,
                   w3_ref, b3_ref, out_ref, *, stile, K, cout):
    g = g_ref[...]
    ow = jnp.dot(nx_ref[0], w1o_ref[...], preferred_element_type=_F32)
    c1 = ow.shape[1]
    owr = jnp.broadcast_to(ow[:, None, :], (stile, K, c1)).reshape(stile * K, c1)
    h = jnp.maximum(jnp.dot(g, w1_ref[...], preferred_element_type=_F32)
                    - owr + b1_ref[...], 0.0)
    h = jnp.maximum(jnp.dot(h, w2_ref[...], preferred_element_type=_F32)
                    + b2_ref[...], 0.0)
    h = jnp.maximum(jnp.dot(h, w3_ref[...], preferred_element_type=_F32)
                    + b3_ref[...], 0.0)
    h = h.reshape(stile, K, cout)
    out_ref[...] = jnp.max(h, axis=1)[None]


def _mlp_pool(gath, new_xyz, ws, bs, B, S, K, stile):
    rows = stile * K
    din = gath.shape[1]
    cout = ws[2].shape[1]
    grid = (B, S // stile)
    nblk = S // stile
    w1o = ws[0][:3]
    full = lambda a: pl.BlockSpec(a.shape, lambda b, t: (0,) * a.ndim)
    b2d = [b.reshape(1, -1) for b in bs]
    out = pl.pallas_call(
        functools.partial(_mlp_pool_body, stile=stile, K=K, cout=cout),
        grid=grid,
        in_specs=[pl.BlockSpec((rows, din), lambda b, t, n=nblk: (b * n + t, 0)),
                  pl.BlockSpec((1, stile, 3), lambda b, t: (b, t, 0)),
                  full(w1o),
                  full(ws[0]), full(b2d[0]), full(ws[1]), full(b2d[1]),
                  full(ws[2]), full(b2d[2])],
        out_specs=pl.BlockSpec((1, stile, cout), lambda b, t: (b, t, 0)),
        out_shape=jax.ShapeDtypeStruct((B, S, cout), _F32),
    )(gath, new_xyz, w1o, ws[0], b2d[0], ws[1], b2d[1], ws[2], b2d[2])
    return out


def _nn3_weights(q, pt):
    M = q.shape[0]
    S = pt.shape[1]
    qx, qy, qz = q[:, 0:1], q[:, 1:2], q[:, 2:3]
    qn = (qx * qx + qy * qy) + qz * qz
    pn = jnp.sum(pt * pt, axis=0, keepdims=True)
    cross = jnp.dot(q, pt)
    d = (qn + pn) - 2.0 * cross
    lio = lax.broadcasted_iota(_I32, (M, S), 1)
    ohs, wts = [], []
    for _ in range(3):
        m = jnp.min(d, axis=1, keepdims=True)
        a = jnp.min(jnp.where(d == m, lio, S), axis=1, keepdims=True)
        oh = lio == a
        wts.append(1.0 / jnp.maximum(m, 1e-10))
        ohs.append(oh)
        d = jnp.where(oh, jnp.float32(np.inf), d)
    wsum = (wts[0] + wts[1]) + wts[2]
    wm = (jnp.where(ohs[0], wts[0] / wsum, 0.0)
          + jnp.where(ohs[1], wts[1] / wsum, 0.0)
          + jnp.where(ohs[2], wts[2] / wsum, 0.0))
    return wm


def _fp1_body(q_ref, pt_ref, f1_ref, f2_ref, w1a_ref, w1b_ref, b1_ref,
              w2_ref, b2_ref, out_ref):
    wm = _nn3_weights(q_ref[0], pt_ref[0])
    interp = jnp.dot(wm, f2_ref[0], preferred_element_type=_F32)
    h = jnp.maximum(
        jnp.dot(f1_ref[0], w1a_ref[...], preferred_element_type=_F32)
        + jnp.dot(interp, w1b_ref[...], preferred_element_type=_F32)
        + b1_ref[...], 0.0)
    h = jnp.maximum(jnp.dot(h, w2_ref[...], preferred_element_type=_F32)
                    + b2_ref[...], 0.0)
    out_ref[...] = h[None]


def _fp1(xyz1, xyz2t, nf1, nf2, ws, bs):
    B, S1, _ = xyz1.shape
    S2 = xyz2t.shape[2]
    c1 = nf1.shape[2]
    c2 = nf2.shape[2]
    w1a, w1b = ws[0][:c1], ws[0][c1:]
    b2d = [b.reshape(1, -1) for b in bs]
    full = lambda a: pl.BlockSpec(a.shape, lambda b: (0,) * a.ndim)
    out = pl.pallas_call(
        _fp1_body,
        grid=(B,),
        in_specs=[pl.BlockSpec((1, S1, 3), lambda b: (b, 0, 0)),
                  pl.BlockSpec((1, 3, S2), lambda b: (b, 0, 0)),
                  pl.BlockSpec((1, S1, c1), lambda b: (b, 0, 0)),
                  pl.BlockSpec((1, S2, c2), lambda b: (b, 0, 0)),
                  full(w1a), full(w1b), full(b2d[0]),
                  full(ws[1]), full(b2d[1])],
        out_specs=pl.BlockSpec((1, S1, ws[1].shape[1]), lambda b: (b, 0, 0)),
        out_shape=jax.ShapeDtypeStruct((B, S1, ws[1].shape[1]), _F32),
    )(xyz1, xyz2t, nf1, nf2, w1a, w1b, b2d[0], ws[1], b2d[1])
    return out


def _fp2fc_body(q_ref, pt_ref, up_ref, w1a_ref, w1b_ref, b1_ref, w2_ref,
                b2_ref, w3_ref, b3_ref, wc1_ref, bc1_ref, wc2_ref, bc2_ref,
                out_ref):
    q = q_ref[0]
    wm = _nn3_weights(q, pt_ref[0])
    interp = jnp.dot(wm, up_ref[0], preferred_element_type=_F32)
    h = jnp.maximum(
        jnp.dot(q, w1a_ref[...], preferred_element_type=_F32)
        + jnp.dot(interp, w1b_ref[...], preferred_element_type=_F32)
        + b1_ref[...], 0.0)
    h = jnp.maximum(jnp.dot(h, w2_ref[...], preferred_element_type=_F32)
                    + b2_ref[...], 0.0)
    h = jnp.maximum(jnp.dot(h, w3_ref[...], preferred_element_type=_F32)
                    + b3_ref[...], 0.0)
    h = jnp.maximum(jnp.dot(h, wc1_ref[...], preferred_element_type=_F32)
                    + bc1_ref[...], 0.0)
    out_ref[...] = (jnp.dot(h, wc2_ref[...], preferred_element_type=_F32)
                    + bc2_ref[...])[None]


def _fp2fc(xyz, xyz1t, up1, fp2_w, fp2_b, fc_w, fc_b, ntile):
    B, N, _ = xyz.shape
    S = xyz1t.shape[2]
    cu = up1.shape[2]
    ncls = fc_w[1].shape[1]
    w1a, w1b = fp2_w[0][:3], fp2_w[0][3:]
    b2d = [b.reshape(1, -1) for b in fp2_b] + [b.reshape(1, -1) for b in fc_b]
    full = lambda a: pl.BlockSpec(a.shape, lambda b, t: (0,) * a.ndim)
    out = pl.pallas_call(
        _fp2fc_body,
        grid=(B, N // ntile),
        in_specs=[pl.BlockSpec((1, ntile, 3), lambda b, t: (b, t, 0)),
                  pl.BlockSpec((1, 3, S), lambda b, t: (b, 0, 0)),
                  pl.BlockSpec((1, S, cu), lambda b, t: (b, 0, 0)),
                  full(w1a), full(w1b), full(b2d[0]),
                  full(fp2_w[1]), full(b2d[1]),
                  full(fp2_w[2]), full(b2d[2]),
                  full(fc_w[0]), full(b2d[3]),
                  full(fc_w[1]), full(b2d[4])],
        out_specs=pl.BlockSpec((1, ntile, ncls), lambda b, t: (b, t, 0)),
        out_shape=jax.ShapeDtypeStruct((B, N, ncls), _F32),
    )(xyz, xyz1t, up1, w1a, w1b, b2d[0], fp2_w[1], b2d[1], fp2_w[2], b2d[2],
      fc_w[0], b2d[3], fc_w[1], b2d[4])
    return out


def kernel(points, sa1_w, sa1_b, sa2_w, sa2_b, fp1_w, fp1_b, fp2_w, fp2_b,
           fc_w, fc_b):
    xyz = points[:, :, :3]
    B, N, _ = xyz.shape
    xt = jnp.transpose(xyz, (0, 2, 1))
    x2, y2, z2 = xt[:, 0], xt[:, 1], xt[:, 2]

    S1, K1 = 512, 32
    _, cx1, cy1, cz1 = _fps(x2, y2, z2, S1)
    new_xyz1 = jnp.stack([cx1, cy1, cz1], axis=-1)
    gidx1 = _ballq(new_xyz1, xt, np.float32(0.2 * 0.2), K1, 128)
    table1 = jnp.concatenate(
        [xyz, xyz, jnp.zeros((B, N, 10), _F32)], axis=-1).reshape(B * N, 16)
    gath1 = _sc_gather_rows(table1, gidx1.reshape(-1))
    w1p = jnp.concatenate([sa1_w[0], jnp.zeros((10, sa1_w[0].shape[1]), _F32)],
                          axis=0)
    nf1 = _mlp_pool(gath1, new_xyz1, [w1p, sa1_w[1], sa1_w[2]], sa1_b,
                    B, S1, K1, 64)

    S2, K2 = 128, 64
    _, cx2, cy2, cz2 = _fps(cx1, cy1, cz1, S2)
    new_xyz2 = jnp.stack([cx2, cy2, cz2], axis=-1)
    xyz1t = jnp.stack([cx1, cy1, cz1], axis=1)
    gidx2 = _ballq(new_xyz2, xyz1t, np.float32(0.4 * 0.4), K2, S2)
    c1 = nf1.shape[2]
    din2 = 3 + c1
    pad2 = (-din2) % 16
    table2 = jnp.concatenate(
        [new_xyz1, nf1, jnp.zeros((B, S1, pad2), _F32)],
        axis=-1).reshape(B * S1, din2 + pad2)
    gath2 = _sc_gather_rows(table2, gidx2.reshape(-1))
    w2p = jnp.concatenate([sa2_w[0], jnp.zeros((pad2, sa2_w[0].shape[1]),
                                               _F32)], axis=0)
    nf2 = _mlp_pool(gath2, new_xyz2, [w2p, sa2_w[1], sa2_w[2]], sa2_b,
                    B, S2, K2, 32)

    xyz2t = jnp.stack([cx2, cy2, cz2], axis=1)
    up1 = _fp1(new_xyz1, xyz2t, nf1, nf2, fp1_w, fp1_b)

    logits = _fp2fc(xyz, xyz1t, up1, fp2_w, fp2_b, fc_w, fc_b, 1024)
    return logits


# speedup vs baseline: 13.6673x; 1.0505x over previous
import functools

import numpy as np
import jax
import jax.numpy as jnp
from jax import lax
from jax.experimental import pallas as pl
from jax.experimental.pallas import tpu as pltpu
from jax.experimental.pallas import tpu_sc as plsc

_F32 = jnp.float32
_I32 = jnp.int32


def _fps_body(x_ref, y_ref, z_ref, idx_ref, cx_ref, cy_ref, cz_ref, *, npoint):
    x = x_ref[...]
    y = y_ref[...]
    z = z_ref[...]
    B, N = x.shape
    lane = lax.broadcasted_iota(_I32, (B, N), 1).astype(_F32)
    piota = lax.broadcasted_iota(_I32, (B, npoint), 1).astype(_F32)

    def body(i, state):
        idx, cxa, cya, cza, dist, far = state
        sel = lane == far
        cx = jnp.sum(jnp.where(sel, x, 0.0), axis=1, keepdims=True)
        cy = jnp.sum(jnp.where(sel, y, 0.0), axis=1, keepdims=True)
        cz = jnp.sum(jnp.where(sel, z, 0.0), axis=1, keepdims=True)
        slot = jnp.where(piota == i.astype(_F32), 1.0, 0.0)
        idx = idx + slot * far
        cxa = cxa + slot * cx
        cya = cya + slot * cy
        cza = cza + slot * cz
        dx = x - cx
        dy = y - cy
        dz = z - cz
        d = (dx * dx + dy * dy) + dz * dz
        dist = jnp.minimum(dist, d)
        m = jnp.max(dist, axis=1, keepdims=True)
        far = jnp.min(jnp.where(dist == m, lane, np.float32(N)),
                      axis=1, keepdims=True)
        return idx, cxa, cya, cza, dist, far

    zp = (piota * 0.0
          + lax.broadcasted_iota(_I32, (B, npoint), 0).astype(_F32) * 0.0)
    zn = (lane * 0.0
          + lax.broadcasted_iota(_I32, (B, N), 0).astype(_F32) * 0.0)
    init = (zp, zp, zp, zp, zn + 1e10, jnp.zeros((B, 1), _F32))
    idx, cxa, cya, cza, _, _ = lax.fori_loop(0, npoint, body, init)
    idx_ref[...] = idx.astype(_I32)
    cx_ref[...] = cxa
    cy_ref[...] = cya
    cz_ref[...] = cza


def _fps(x, y, z, npoint):
    B, N = x.shape
    full = lambda s: pl.BlockSpec(s, lambda: (0,) * len(s))
    outs = pl.pallas_call(
        functools.partial(_fps_body, npoint=npoint),
        in_specs=[full((B, N))] * 3,
        out_specs=[full((B, npoint))] * 4,
        out_shape=[jax.ShapeDtypeStruct((B, npoint), _I32)] +
                  [jax.ShapeDtypeStruct((B, npoint), _F32)] * 3,
    )(x, y, z)
    return outs


def _cumsum_lanes(m):
    c = m
    sh = 1
    n = m.shape[-1]
    while sh < n:
        z = jnp.zeros(m.shape[:-1] + (sh,), m.dtype)
        c = c + jnp.concatenate([z, c[..., :-sh]], axis=-1)
        sh *= 2
    return c


def _ballq_body(q_ref, pt_ref, out_ref, *, thr, K, N):
    b = pl.program_id(0)
    q = q_ref[0]
    pt = pt_ref[0]
    qx = q[:, 0:1]
    qy = q[:, 1:2]
    qz = q[:, 2:3]
    qn = (qx * qx + qy * qy) + qz * qz
    pn = jnp.sum(pt * pt, axis=0, keepdims=True)
    cross = jnp.dot(q, pt)
    sqd = (qn + pn) - 2.0 * cross
    mask = jnp.where(sqd > thr, 0.0, 1.0)
    c = _cumsum_lanes(mask)
    cnt = c[:, N - 1:N]
    svals = [jnp.sum(jnp.minimum(c, np.float32(t)), axis=1, keepdims=True)
             for t in range(K + 1)]
    cols = [np.float32(N) - (svals[k + 1] - svals[k]) for k in range(K)]
    pk = jnp.concatenate(cols, axis=1)
    ST = q.shape[0]
    kio = lax.broadcasted_iota(_I32, (ST, K), 1)
    grp = jnp.where(kio < cnt.astype(_I32), pk, cols[0])
    out_ref[...] = (grp.astype(_I32) + b * N)[None]


def _ballq(q, pt, thr, K, stile):
    B, S, _ = q.shape
    N = pt.shape[-1]
    grid = (B, S // stile)
    out = pl.pallas_call(
        functools.partial(_ballq_body, thr=thr, K=K, N=N),
        grid=grid,
        in_specs=[pl.BlockSpec((1, stile, 3), lambda b, t: (b, t, 0)),
                  pl.BlockSpec((1, 3, N), lambda b, t: (b, 0, 0))],
        out_specs=pl.BlockSpec((1, stile, K), lambda b, t: (b, t, 0)),
        out_shape=jax.ShapeDtypeStruct((B, S, K), _I32),
    )(q, pt)
    return out


@functools.lru_cache(maxsize=None)
def _make_sc_gather(V, D, Btot):
    NC, NS = 2, 16
    NW = NC * NS
    CH = 128
    bw = Btot // NW
    nch = bw // CH
    mesh = plsc.VectorSubcoreMesh(core_axis_name="c", subcore_axis_name="s")

    @functools.partial(
        pl.kernel,
        mesh=mesh,
        out_type=jax.ShapeDtypeStruct((Btot, D), _F32),
        scratch_types=[pltpu.VMEM((CH,), _I32),
                       pltpu.VMEM((CH, D), _F32),
                       pltpu.SemaphoreType.DMA],
        compiler_params=pltpu.CompilerParams(use_tc_tiling_on_sc=False),
    )
    def k(table_hbm, idx_hbm, out_hbm, idx_v, rows_v, sem):
        wid = lax.axis_index("s") * NC + lax.axis_index("c")
        base = wid * bw

        def chunk(j, carry):
            off = base + j * CH
            pltpu.sync_copy(idx_hbm.at[pl.ds(off, CH)], idx_v)
            pltpu.async_copy(table_hbm.at[idx_v], rows_v, sem).wait()
            pltpu.sync_copy(rows_v, out_hbm.at[pl.ds(off, CH)])
            return carry

        lax.fori_loop(0, nch, chunk, 0)

    return k


def _sc_gather_rows(table, idx):
    V, D = table.shape
    return _make_sc_gather(V, D, idx.shape[0])(table, idx)


def _mlp_pool_body(g_ref, nx_ref, w1o_ref, w1_ref, b1_ref, w2_ref, b2_ref,
                   w3_ref, b3_ref, out_ref, *, stile, K, cout):
    g = g_ref[...]
    ow = jnp.dot(nx_ref[0], w1o_ref[...], preferred_element_type=_F32)
    c1 = ow.shape[1]
    owr = jnp.broadcast_to(ow[:, None, :], (stile, K, c1)).reshape(stile * K, c1)
    h = jnp.maximum(jnp.dot(g, w1_ref[...], preferred_element_type=_F32)
                    - owr + b1_ref[...], 0.0)
    h = jnp.maximum(jnp.dot(h, w2_ref[...], preferred_element_type=_F32)
                    + b2_ref[...], 0.0)
    h = jnp.maximum(jnp.dot(h, w3_ref[...], preferred_element_type=_F32)
                    + b3_ref[...], 0.0)
    h = h.reshape(stile, K, cout)
    out_ref[...] = jnp.max(h, axis=1)[None]


def _mlp_pool(gath, new_xyz, ws, bs, B, S, K, stile):
    rows = stile * K
    din = gath.shape[1]
    cout = ws[2].shape[1]
    grid = (B, S // stile)
    nblk = S // stile
    w1o = ws[0][:3]
    full = lambda a: pl.BlockSpec(a.shape, lambda b, t: (0,) * a.ndim)
    b2d = [b.reshape(1, -1) for b in bs]
    out = pl.pallas_call(
        functools.partial(_mlp_pool_body, stile=stile, K=K, cout=cout),
        grid=grid,
        in_specs=[pl.BlockSpec((rows, din), lambda b, t, n=nblk: (b * n + t, 0)),
                  pl.BlockSpec((1, stile, 3), lambda b, t: (b, t, 0)),
                  full(w1o),
                  full(ws[0]), full(b2d[0]), full(ws[1]), full(b2d[1]),
                  full(ws[2]), full(b2d[2])],
        out_specs=pl.BlockSpec((1, stile, cout), lambda b, t: (b, t, 0)),
        out_shape=jax.ShapeDtypeStruct((B, S, cout), _F32),
    )(gath, new_xyz, w1o, ws[0], b2d[0], ws[1], b2d[1], ws[2], b2d[2])
    return out


def _nn3_weights(q, pt):
    M = q.shape[0]
    S = pt.shape[1]
    qx, qy, qz = q[:, 0:1], q[:, 1:2], q[:, 2:3]
    qn = (qx * qx + qy * qy) + qz * qz
    pn = jnp.sum(pt * pt, axis=0, keepdims=True)
    cross = jnp.dot(q, pt)
    d = (qn + pn) - 2.0 * cross
    lio = lax.broadcasted_iota(_I32, (M, S), 1)
    ohs, wts = [], []
    for _ in range(3):
        m = jnp.min(d, axis=1, keepdims=True)
        a = jnp.min(jnp.where(d == m, lio, S), axis=1, keepdims=True)
        oh = lio == a
        wts.append(1.0 / jnp.maximum(m, 1e-10))
        ohs.append(oh)
        d = jnp.where(oh, jnp.float32(np.inf), d)
    wsum = (wts[0] + wts[1]) + wts[2]
    wm = (jnp.where(ohs[0], wts[0] / wsum, 0.0)
          + jnp.where(ohs[1], wts[1] / wsum, 0.0)
          + jnp.where(ohs[2], wts[2] / wsum, 0.0))
    return wm


def _fp1_body(q_ref, pt_ref, f1_ref, f2_ref, w1a_ref, w1b_ref, b1_ref,
              w2_ref, b2_ref, out_ref):
    wm = _nn3_weights(q_ref[0], pt_ref[0])
    interp = jnp.dot(wm, f2_ref[0], preferred_element_type=_F32)
    h = jnp.maximum(
        jnp.dot(f1_ref[0], w1a_ref[...], preferred_element_type=_F32)
        + jnp.dot(interp, w1b_ref[...], preferred_element_type=_F32)
        + b1_ref[...], 0.0)
    h = jnp.maximum(jnp.dot(h, w2_ref[...], preferred_element_type=_F32)
                    + b2_ref[...], 0.0)
    out_ref[...] = h[None]


def _fp1(xyz1, xyz2t, nf1, nf2, ws, bs):
    B, S1, _ = xyz1.shape
    S2 = xyz2t.shape[2]
    c1 = nf1.shape[2]
    c2 = nf2.shape[2]
    w1a, w1b = ws[0][:c1], ws[0][c1:]
    b2d = [b.reshape(1, -1) for b in bs]
    full = lambda a: pl.BlockSpec(a.shape, lambda b: (0,) * a.ndim)
    out = pl.pallas_call(
        _fp1_body,
        grid=(B,),
        in_specs=[pl.BlockSpec((1, S1, 3), lambda b: (b, 0, 0)),
                  pl.BlockSpec((1, 3, S2), lambda b: (b, 0, 0)),
                  pl.BlockSpec((1, S1, c1), lambda b: (b, 0, 0)),
                  pl.BlockSpec((1, S2, c2), lambda b: (b, 0, 0)),
                  full(w1a), full(w1b), full(b2d[0]),
                  full(ws[1]), full(b2d[1])],
        out_specs=pl.BlockSpec((1, S1, ws[1].shape[1]), lambda b: (b, 0, 0)),
        out_shape=jax.ShapeDtypeStruct((B, S1, ws[1].shape[1]), _F32),
    )(xyz1, xyz2t, nf1, nf2, w1a, w1b, b2d[0], ws[1], b2d[1])
    return out


def _fp2fc_body(q_ref, pt_ref, up_ref, w1a_ref, w1b_ref, b1_ref, w2_ref,
                b2_ref, w3_ref, b3_ref, wc1_ref, bc1_ref, wc2_ref, bc2_ref,
                out_ref):
    q = q_ref[0]
    wm = _nn3_weights(q, pt_ref[0])
    interp = jnp.dot(wm, up_ref[0], preferred_element_type=_F32)
    h = jnp.maximum(
        jnp.dot(q, w1a_ref[...], preferred_element_type=_F32)
        + jnp.dot(interp, w1b_ref[...], preferred_element_type=_F32)
        + b1_ref[...], 0.0)
    h = jnp.maximum(jnp.dot(h, w2_ref[...], preferred_element_type=_F32)
                    + b2_ref[...], 0.0)
    h = jnp.maximum(jnp.dot(h, w3_ref[...], preferred_element_type=_F32)
                    + b3_ref[...], 0.0)
    h = jnp.maximum(jnp.dot(h, wc1_ref[...], preferred_element_type=_F32)
                    + bc1_ref[...], 0.0)
    out_ref[...] = (jnp.dot(h, wc2_ref[...], preferred_element_type=_F32)
                    + bc2_ref[...])[None]


def _fp2fc(xyz, xyz1t, up1, fp2_w, fp2_b, fc_w, fc_b, ntile):
    B, N, _ = xyz.shape
    S = xyz1t.shape[2]
    cu = up1.shape[2]
    ncls = fc_w[1].shape[1]
    w1a, w1b = fp2_w[0][:3], fp2_w[0][3:]
    b2d = [b.reshape(1, -1) for b in fp2_b] + [b.reshape(1, -1) for b in fc_b]
    full = lambda a: pl.BlockSpec(a.shape, lambda b, t: (0,) * a.ndim)
    out = pl.pallas_call(
        _fp2fc_body,
        grid=(B, N // ntile),
        in_specs=[pl.BlockSpec((1, ntile, 3), lambda b, t: (b, t, 0)),
                  pl.BlockSpec((1, 3, S), lambda b, t: (b, 0, 0)),
                  pl.BlockSpec((1, S, cu), lambda b, t: (b, 0, 0)),
                  full(w1a), full(w1b), full(b2d[0]),
                  full(fp2_w[1]), full(b2d[1]),
                  full(fp2_w[2]), full(b2d[2]),
                  full(fc_w[0]), full(b2d[3]),
                  full(fc_w[1]), full(b2d[4])],
        out_specs=pl.BlockSpec((1, ntile, ncls), lambda b, t: (b, t, 0)),
        out_shape=jax.ShapeDtypeStruct((B, N, ncls), _F32),
    )(xyz, xyz1t, up1, w1a, w1b, b2d[0], fp2_w[1], b2d[1], fp2_w[2], b2d[2],
      fc_w[0], b2d[3], fc_w[1], b2d[4])
    return out


def kernel(points, sa1_w, sa1_b, sa2_w, sa2_b, fp1_w, fp1_b, fp2_w, fp2_b,
           fc_w, fc_b):
    xyz = points[:, :, :3]
    B, N, _ = xyz.shape
    xt = jnp.transpose(xyz, (0, 2, 1))
    x2, y2, z2 = xt[:, 0], xt[:, 1], xt[:, 2]

    S1, K1 = 512, 32
    _, cx1, cy1, cz1 = _fps(x2, y2, z2, S1)
    new_xyz1 = jnp.stack([cx1, cy1, cz1], axis=-1)
    gidx1 = _ballq(new_xyz1, xt, np.float32(0.2 * 0.2), K1, 128)
    table1 = jnp.concatenate(
        [xyz, xyz, jnp.zeros((B, N, 10), _F32)], axis=-1).reshape(B * N, 16)
    gath1 = _sc_gather_rows(table1, gidx1.reshape(-1))
    w1p = jnp.concatenate([sa1_w[0], jnp.zeros((10, sa1_w[0].shape[1]), _F32)],
                          axis=0)
    nf1 = _mlp_pool(gath1, new_xyz1, [w1p, sa1_w[1], sa1_w[2]], sa1_b,
                    B, S1, K1, 64)

    S2, K2 = 128, 64
    _, cx2, cy2, cz2 = _fps(cx1, cy1, cz1, S2)
    new_xyz2 = jnp.stack([cx2, cy2, cz2], axis=-1)
    xyz1t = jnp.stack([cx1, cy1, cz1], axis=1)
    gidx2 = _ballq(new_xyz2, xyz1t, np.float32(0.4 * 0.4), K2, S2)
    c1 = nf1.shape[2]
    din2 = 3 + c1
    pad2 = (-din2) % 16
    table2 = jnp.concatenate(
        [new_xyz1, nf1, jnp.zeros((B, S1, pad2), _F32)],
        axis=-1).reshape(B * S1, din2 + pad2)
    gath2 = _sc_gather_rows(table2, gidx2.reshape(-1))
    w2p = jnp.concatenate([sa2_w[0], jnp.zeros((pad2, sa2_w[0].shape[1]),
                                               _F32)], axis=0)
    nf2 = _mlp_pool(gath2, new_xyz2, [w2p, sa2_w[1], sa2_w[2]], sa2_b,
                    B, S2, K2, 32)

    xyz2t = jnp.stack([cx2, cy2, cz2], axis=1)
    up1 = _fp1(new_xyz1, xyz2t, nf1, nf2, fp1_w, fp1_b)

    logits = _fp2fc(xyz, xyz1t, up1, fp2_w, fp2_b, fc_w, fc_b, 1024)
    return logits
